# baseline (speedup 1.0000x reference)
import jax
import jax.numpy as jnp
import numpy as np
from jax.experimental import pallas as pl
from jax.experimental.pallas import tpu as pltpu

KH, KW = 3, 3
BN_EPS = 1e-5
BBLK1 = 32
BBLK2 = 32
WCH = 4


def _conv_parts(x_ref, w_ref):
    BB, H, WC = x_ref.shape
    KD, ND = w_ref.shape
    CL = KD // KH
    NCH = WC // CL
    x = x_ref[...].astype(jnp.bfloat16)
    zrow = jnp.zeros((BB, 1, WC), x.dtype)
    xu = jnp.concatenate([x[:, 1:, :], zrow], axis=1)
    xd = jnp.concatenate([zrow, x[:, :H - 1, :]], axis=1)
    parts = []
    for c in range(NCH):
        sl = slice(c * CL, (c + 1) * CL)
        parts.append(jnp.concatenate([xu[:, :, sl], x[:, :, sl], xd[:, :, sl]],
                                     axis=2))
    lhs = jnp.stack(parts, axis=0).reshape(NCH * BB * H, KD)
    return jnp.dot(lhs, w_ref[...], preferred_element_type=jnp.float32)


def _stats_kernel(x_ref, w_ref, sel_ref, stats_ref):
    BB, H, _ = x_ref.shape
    part = _conv_parts(x_ref, w_ref)
    ND = part.shape[1]
    R = BB * H
    NCH = part.shape[0] // R
    CO = sel_ref.shape[1]
    psum = jnp.sum(part, axis=0, keepdims=True)
    psq = jnp.sum(part * part, axis=0, keepdims=True)
    st = jnp.concatenate([psum, psq], axis=0)
    stats = jnp.dot(st, sel_ref[...],
                    preferred_element_type=jnp.float32,
                    precision=jax.lax.Precision.HIGHEST)
    a = part[:(NCH - 1) * R, ND - CO:ND]
    b = part[R:, 0:CO]
    corr = 2.0 * jnp.sum(a * b, axis=0, keepdims=True)
    stats_ref[...] = stats + jnp.concatenate(
        [jnp.zeros((1, CO), jnp.float32), corr], axis=0)


def _conv_bn_elu_kernel(x_ref, w_ref, scale_ref, shift_ref, o_ref):
    BB, H, _ = x_ref.shape
    part = _conv_parts(x_ref, w_ref)
    ND = part.shape[1]
    R = BB * H
    NCH = part.shape[0] // R
    CO = ND // (2 * WCH + 1)
    body = ND - CO
    pieces = [part[0:R, 0:body]]
    for c in range(1, NCH):
        prev = part[(c - 1) * R:c * R, body:ND]
        cur = part[c * R:(c + 1) * R, :]
        pieces.append(prev + cur[:, 0:CO])
        pieces.append(cur[:, CO:body])
    pieces.append(part[(NCH - 1) * R:, body:ND])
    acc = jnp.concatenate(pieces, axis=1)
    v = acc * scale_ref[...] + shift_ref[...]
    e = jnp.where(v > 0, v, jnp.exp(jnp.minimum(v, 0.0)) - 1.0)
    o_ref[...] = e.reshape(BB, H, acc.shape[1]).astype(o_ref.dtype)


def _chunk_weights(weight):
    Cin, Cout = weight.shape[0], weight.shape[1]
    Jo = 2 * WCH + 1
    w_t = jnp.transpose(weight, (2, 3, 0, 1)).astype(jnp.float32)
    P = np.zeros((KW, WCH, Jo), np.float32)
    jw = np.arange(WCH)
    for kw in range(KW):
        P[kw, jw, 2 * jw + kw] = 1.0
    band = jnp.einsum('kwo,hkic->hwioc', jnp.asarray(P), w_t)
    return band.reshape(KH * WCH * Cin, Jo * Cout)


@jax.jit
def _forward(x_nchw, weight, bias, gamma, beta):
    B, Cin, H, W = x_nchw.shape
    Cout = weight.shape[1]
    Wo = 2 * W + 1
    M = Wo * Cout
    nB1 = B // BBLK1
    nB2 = B // BBLK2

    x_rows = jnp.transpose(x_nchw, (0, 2, 3, 1)).reshape(B, H, W * Cin)
    wc = _chunk_weights(weight).astype(jnp.bfloat16)
    KD, ND = wc.shape
    sel = jnp.tile(jnp.eye(Cout, dtype=jnp.float32), (ND // Cout, 1))

    stats = pl.pallas_call(
        _stats_kernel,
        out_shape=jax.ShapeDtypeStruct((nB1, 2, Cout), jnp.float32),
        grid=(nB1,),
        in_specs=[
            pl.BlockSpec((BBLK1, H, W * Cin), lambda b: (b, 0, 0)),
            pl.BlockSpec((KD, ND), lambda b: (0, 0)),
            pl.BlockSpec((ND, Cout), lambda b: (0, 0)),
        ],
        out_specs=pl.BlockSpec((None, 2, Cout), lambda b: (b, 0, 0)),
        compiler_params=pltpu.CompilerParams(dimension_semantics=("parallel",)),
    )(x_rows, wc, sel)

    cnt = float(B * H * Wo)
    s = jnp.sum(stats[:, 0, :], axis=0)
    ss = jnp.sum(stats[:, 1, :], axis=0)
    mean_acc = s / cnt
    var = ss / cnt - mean_acc * mean_acc
    inv = jax.lax.rsqrt(var + BN_EPS)
    scale = gamma.astype(jnp.float32) * inv
    shift2 = beta.astype(jnp.float32) - mean_acc * scale
    scale_m = jnp.tile(scale, Wo).reshape(1, M)
    shift_m = jnp.tile(shift2, Wo).reshape(1, M)

    out_flat = pl.pallas_call(
        _conv_bn_elu_kernel,
        out_shape=jax.ShapeDtypeStruct((B, H, M), jnp.bfloat16),
        grid=(nB2,),
        in_specs=[
            pl.BlockSpec((BBLK2, H, W * Cin), lambda b: (b, 0, 0)),
            pl.BlockSpec((KD, ND), lambda b: (0, 0)),
            pl.BlockSpec((1, M), lambda b: (0, 0)),
            pl.BlockSpec((1, M), lambda b: (0, 0)),
        ],
        out_specs=pl.BlockSpec((BBLK2, H, M), lambda b: (b, 0, 0)),
        compiler_params=pltpu.CompilerParams(dimension_semantics=("parallel",)),
    )(x_rows, wc, scale_m, shift_m)

    out_t = jnp.transpose(out_flat.reshape(B, H, Wo, Cout), (0, 3, 1, 2))
    out_t = jax.lax.optimization_barrier(out_t)
    return out_t.astype(jnp.float32)


def kernel(x_nchw, weight, bias, gamma, beta):
    return _forward(x_nchw, weight, bias, gamma, beta)


# speedup vs baseline: 1.0482x; 1.0482x over previous
import jax
import jax.numpy as jnp
import numpy as np
from jax.experimental import pallas as pl
from jax.experimental.pallas import tpu as pltpu

KH, KW = 3, 3
BN_EPS = 1e-5
BBLK1 = 32
BBLK2 = 32
WCH = 4


def _conv_parts(x_ref, w_ref):
    BB, H, WC = x_ref.shape
    KD, ND = w_ref.shape
    CL = KD // KH
    NCH = WC // CL
    x = x_ref[...]
    zrow = jnp.zeros((BB, 1, WC), x.dtype)
    xu = jnp.concatenate([x[:, 1:, :], zrow], axis=1)
    xd = jnp.concatenate([zrow, x[:, :H - 1, :]], axis=1)
    parts = []
    for c in range(NCH):
        sl = slice(c * CL, (c + 1) * CL)
        parts.append(jnp.concatenate([xu[:, :, sl], x[:, :, sl], xd[:, :, sl]],
                                     axis=2))
    lhs = jnp.stack(parts, axis=0).reshape(NCH * BB * H, KD)
    return jnp.dot(lhs, w_ref[...], preferred_element_type=jnp.float32)


def _stats_kernel(x_ref, w_ref, sel_ref, stats_ref):
    BB, H, _ = x_ref.shape
    part = _conv_parts(x_ref, w_ref)
    ND = part.shape[1]
    R = BB * H
    NCH = part.shape[0] // R
    CO = sel_ref.shape[1]
    psum = jnp.sum(part, axis=0, keepdims=True)
    psq = jnp.sum(part * part, axis=0, keepdims=True)
    st = jnp.concatenate([psum, psq], axis=0)
    stats = jnp.dot(st, sel_ref[...],
                    preferred_element_type=jnp.float32,
                    precision=jax.lax.Precision.HIGHEST)
    a = part[:(NCH - 1) * R, ND - CO:ND]
    b = part[R:, 0:CO]
    corr = 2.0 * jnp.sum(a * b, axis=0, keepdims=True)
    stats_ref[...] = stats + jnp.concatenate(
        [jnp.zeros((1, CO), jnp.float32), corr], axis=0)


def _conv_bn_elu_kernel(x_ref, w_ref, scale_ref, shift_ref, o_ref):
    BB, H, _ = x_ref.shape
    part = _conv_parts(x_ref, w_ref)
    ND = part.shape[1]
    R = BB * H
    NCH = part.shape[0] // R
    CO = ND // (2 * WCH + 1)
    body = ND - CO
    pieces = [part[0:R, 0:body]]
    for c in range(1, NCH):
        prev = part[(c - 1) * R:c * R, body:ND]
        cur = part[c * R:(c + 1) * R, :]
        pieces.append(prev + cur[:, 0:CO])
        pieces.append(cur[:, CO:body])
    pieces.append(part[(NCH - 1) * R:, body:ND])
    acc = jnp.concatenate(pieces, axis=1)
    v = acc * scale_ref[...] + shift_ref[...]
    e = jnp.where(v > 0, v, jnp.exp(v) - 1.0)
    o_ref[...] = e.reshape(BB, H, acc.shape[1]).astype(o_ref.dtype)


def _chunk_weights(weight):
    Cin, Cout = weight.shape[0], weight.shape[1]
    Jo = 2 * WCH + 1
    w_t = jnp.transpose(weight, (2, 3, 0, 1)).astype(jnp.float32)
    P = np.zeros((KW, WCH, Jo), np.float32)
    jw = np.arange(WCH)
    for kw in range(KW):
        P[kw, jw, 2 * jw + kw] = 1.0
    band = jnp.einsum('kwo,hkic->hwioc', jnp.asarray(P), w_t)
    return band.reshape(KH * WCH * Cin, Jo * Cout)


@jax.jit
def _forward(x_nchw, weight, bias, gamma, beta):
    B, Cin, H, W = x_nchw.shape
    Cout = weight.shape[1]
    Wo = 2 * W + 1
    M = Wo * Cout
    nB1 = B // BBLK1
    nB2 = B // BBLK2

    x_bf = jax.lax.optimization_barrier(x_nchw.astype(jnp.bfloat16))
    x_rows = jnp.transpose(x_bf, (0, 2, 3, 1)).reshape(B, H, W * Cin)
    wc = _chunk_weights(weight).astype(jnp.bfloat16)
    KD, ND = wc.shape
    sel = jnp.tile(jnp.eye(Cout, dtype=jnp.float32), (ND // Cout, 1))

    stats = pl.pallas_call(
        _stats_kernel,
        out_shape=jax.ShapeDtypeStruct((nB1, 2, Cout), jnp.float32),
        grid=(nB1,),
        in_specs=[
            pl.BlockSpec((BBLK1, H, W * Cin), lambda b: (b, 0, 0)),
            pl.BlockSpec((KD, ND), lambda b: (0, 0)),
            pl.BlockSpec((ND, Cout), lambda b: (0, 0)),
        ],
        out_specs=pl.BlockSpec((None, 2, Cout), lambda b: (b, 0, 0)),
        compiler_params=pltpu.CompilerParams(dimension_semantics=("parallel",)),
    )(x_rows, wc, sel)

    cnt = float(B * H * Wo)
    s = jnp.sum(stats[:, 0, :], axis=0)
    ss = jnp.sum(stats[:, 1, :], axis=0)
    mean_acc = s / cnt
    var = ss / cnt - mean_acc * mean_acc
    inv = jax.lax.rsqrt(var + BN_EPS)
    scale = gamma.astype(jnp.float32) * inv
    shift2 = beta.astype(jnp.float32) - mean_acc * scale
    scale_m = jnp.tile(scale, Wo).reshape(1, M)
    shift_m = jnp.tile(shift2, Wo).reshape(1, M)

    out_flat = pl.pallas_call(
        _conv_bn_elu_kernel,
        out_shape=jax.ShapeDtypeStruct((B, H, M), jnp.bfloat16),
        grid=(nB2,),
        in_specs=[
            pl.BlockSpec((BBLK2, H, W * Cin), lambda b: (b, 0, 0)),
            pl.BlockSpec((KD, ND), lambda b: (0, 0)),
            pl.BlockSpec((1, M), lambda b: (0, 0)),
            pl.BlockSpec((1, M), lambda b: (0, 0)),
        ],
        out_specs=pl.BlockSpec((BBLK2, H, M), lambda b: (b, 0, 0)),
        compiler_params=pltpu.CompilerParams(dimension_semantics=("parallel",)),
    )(x_rows, wc, scale_m, shift_m)

    out_t = jnp.transpose(out_flat.reshape(B, H, Wo, Cout), (0, 3, 1, 2))
    out_t = jax.lax.optimization_barrier(out_t)
    return out_t.astype(jnp.float32)


def kernel(x_nchw, weight, bias, gamma, beta):
    return _forward(x_nchw, weight, bias, gamma, beta)


# speedup vs baseline: 1.0576x; 1.0090x over previous
import jax
import jax.numpy as jnp
import numpy as np
from jax.experimental import pallas as pl
from jax.experimental.pallas import tpu as pltpu

KH, KW = 3, 3
BN_EPS = 1e-5
BBLK1 = 64
BBLK2 = 32
WCH = 4


def _conv_parts(x_ref, w_ref):
    BB, H, WC = x_ref.shape
    KD, ND = w_ref.shape
    CL = KD // KH
    NCH = WC // CL
    x = x_ref[...]
    zrow = jnp.zeros((BB, 1, WC), x.dtype)
    xu = jnp.concatenate([x[:, 1:, :], zrow], axis=1)
    xd = jnp.concatenate([zrow, x[:, :H - 1, :]], axis=1)
    parts = []
    for c in range(NCH):
        sl = slice(c * CL, (c + 1) * CL)
        parts.append(jnp.concatenate([xu[:, :, sl], x[:, :, sl], xd[:, :, sl]],
                                     axis=2))
    lhs = jnp.stack(parts, axis=0).reshape(NCH * BB * H, KD)
    return jnp.dot(lhs, w_ref[...], preferred_element_type=jnp.float32)


def _stats_kernel(x_ref, w_ref, sel_ref, stats_ref):
    BB, H, _ = x_ref.shape
    part = _conv_parts(x_ref, w_ref)
    ND = part.shape[1]
    R = BB * H
    NCH = part.shape[0] // R
    CO = sel_ref.shape[1]
    psum = jnp.sum(part, axis=0, keepdims=True)
    psq = jnp.sum(part * part, axis=0, keepdims=True)
    st = jnp.concatenate([psum, psq], axis=0)
    stats = jnp.dot(st, sel_ref[...],
                    preferred_element_type=jnp.float32,
                    precision=jax.lax.Precision.HIGHEST)
    a = part[:(NCH - 1) * R, ND - CO:ND]
    b = part[R:, 0:CO]
    corr = 2.0 * jnp.sum(a * b, axis=0, keepdims=True)
    stats_ref[...] = stats + jnp.concatenate(
        [jnp.zeros((1, CO), jnp.float32), corr], axis=0)


def _conv_bn_elu_kernel(x_ref, w_ref, scale_ref, shift_ref, o_ref):
    BB, H, _ = x_ref.shape
    part = _conv_parts(x_ref, w_ref)
    ND = part.shape[1]
    R = BB * H
    NCH = part.shape[0] // R
    CO = ND // (2 * WCH + 1)
    body = ND - CO
    pieces = [part[0:R, 0:body]]
    for c in range(1, NCH):
        prev = part[(c - 1) * R:c * R, body:ND]
        cur = part[c * R:(c + 1) * R, :]
        pieces.append(prev + cur[:, 0:CO])
        pieces.append(cur[:, CO:body])
    pieces.append(part[(NCH - 1) * R:, body:ND])
    acc = jnp.concatenate(pieces, axis=1)
    v = acc * scale_ref[...] + shift_ref[...]
    e = jnp.where(v > 0, v, jnp.exp(v) - 1.0)
    o_ref[...] = e.reshape(BB, H, acc.shape[1]).astype(o_ref.dtype)


def _chunk_weights(weight):
    Cin, Cout = weight.shape[0], weight.shape[1]
    Jo = 2 * WCH + 1
    w_t = jnp.transpose(weight, (2, 3, 0, 1)).astype(jnp.float32)
    P = np.zeros((KW, WCH, Jo), np.float32)
    jw = np.arange(WCH)
    for kw in range(KW):
        P[kw, jw, 2 * jw + kw] = 1.0
    band = jnp.einsum('kwo,hkic->hwioc', jnp.asarray(P), w_t)
    return band.reshape(KH * WCH * Cin, Jo * Cout)


@jax.jit
def _forward(x_nchw, weight, bias, gamma, beta):
    B, Cin, H, W = x_nchw.shape
    Cout = weight.shape[1]
    Wo = 2 * W + 1
    M = Wo * Cout
    nB1 = B // BBLK1
    nB2 = B // BBLK2

    x_bf = jax.lax.optimization_barrier(x_nchw.astype(jnp.bfloat16))
    x_rows = jnp.transpose(x_bf, (0, 2, 3, 1)).reshape(B, H, W * Cin)
    wc = _chunk_weights(weight).astype(jnp.bfloat16)
    KD, ND = wc.shape
    sel = jnp.tile(jnp.eye(Cout, dtype=jnp.float32), (ND // Cout, 1))

    stats = pl.pallas_call(
        _stats_kernel,
        out_shape=jax.ShapeDtypeStruct((nB1, 2, Cout), jnp.float32),
        grid=(nB1,),
        in_specs=[
            pl.BlockSpec((BBLK1, H, W * Cin), lambda b: (b, 0, 0)),
            pl.BlockSpec((KD, ND), lambda b: (0, 0)),
            pl.BlockSpec((ND, Cout), lambda b: (0, 0)),
        ],
        out_specs=pl.BlockSpec((None, 2, Cout), lambda b: (b, 0, 0)),
        compiler_params=pltpu.CompilerParams(dimension_semantics=("parallel",)),
    )(x_rows, wc, sel)

    cnt = float(B * H * Wo)
    s = jnp.sum(stats[:, 0, :], axis=0)
    ss = jnp.sum(stats[:, 1, :], axis=0)
    mean_acc = s / cnt
    var = ss / cnt - mean_acc * mean_acc
    inv = jax.lax.rsqrt(var + BN_EPS)
    scale = gamma.astype(jnp.float32) * inv
    shift2 = beta.astype(jnp.float32) - mean_acc * scale
    scale_m = jnp.tile(scale, Wo).reshape(1, M)
    shift_m = jnp.tile(shift2, Wo).reshape(1, M)

    out_flat = pl.pallas_call(
        _conv_bn_elu_kernel,
        out_shape=jax.ShapeDtypeStruct((B, H, M), jnp.bfloat16),
        grid=(nB2,),
        in_specs=[
            pl.BlockSpec((BBLK2, H, W * Cin), lambda b: (b, 0, 0)),
            pl.BlockSpec((KD, ND), lambda b: (0, 0)),
            pl.BlockSpec((1, M), lambda b: (0, 0)),
            pl.BlockSpec((1, M), lambda b: (0, 0)),
        ],
        out_specs=pl.BlockSpec((BBLK2, H, M), lambda b: (b, 0, 0)),
        compiler_params=pltpu.CompilerParams(dimension_semantics=("parallel",)),
    )(x_rows, wc, scale_m, shift_m)

    out_t = jnp.transpose(out_flat.reshape(B, H, Wo, Cout), (0, 3, 1, 2))
    out_t = jax.lax.optimization_barrier(out_t)
    return out_t.astype(jnp.float32)


def kernel(x_nchw, weight, bias, gamma, beta):
    return _forward(x_nchw, weight, bias, gamma, beta)
